# Initial kernel scaffold; baseline (speedup 1.0000x reference)
import jax
import jax.numpy as jnp
from jax.experimental import pallas as pl


def kernel(fwd_x, fwd_edge_index, fwd_edge_attr, bwd_x, bwd_edge_index, bwd_edge_attr, embed, Wih_f, Whh_f, bih_f, bhh_f, Wih_r, Whh_r, bih_r, bhh_r, lin1_W, lin1_b, lin2_W, lin2_b, fgat_Wl, fgat_bl, fgat_Wr, fgat_br, fgat_att, fgat_bias, bgat_Wl, bgat_bl, bgat_Wr, bgat_br, bgat_att, bgat_bias):
    raise NotImplementedError("write your pallas kernel here")


# speedup vs baseline: 9.4528x; 9.4528x over previous
import functools

import jax
import jax.numpy as jnp
from jax.experimental import pallas as pl
from jax.experimental.pallas import tpu as pltpu

MAX_STATES = 50
TID = MAX_STATES + 3
REGEX_IDX = TID + 2 + TID + TID
HID = REGEX_IDX + TID
N_NODES = 10000
N_EDGES = 160000
SEQ_LEN = 8
VOCAB = 100
EMB = 32
LSTM = 64
H4 = 4 * LSTM

EB = 2560
E2 = 2 * N_EDGES
NBLK = E2 // EB


def _edge_score_body(tok_ref, embT_ref, wihf_ref, whhf_ref, bf_ref,
                     wihr_ref, br_ref, l1w_ref, l1b_ref, l2w_ref, l2b_ref,
                     out_ref):
    tok = tok_ref[...]
    embT = embT_ref[...]
    Af = jnp.dot(wihf_ref[...], embT, preferred_element_type=jnp.float32)
    Whh = whhf_ref[...]
    bf = bf_ref[...]

    def onehot(row):
        i = jax.lax.broadcasted_iota(jnp.int32, (VOCAB, EB), 0)
        return (i == row[None, :]).astype(jnp.float32)

    h = jnp.zeros((LSTM, EB), jnp.float32)
    c = jnp.zeros((LSTM, EB), jnp.float32)
    for t in range(SEQ_LEN):
        oh = onehot(tok[t])
        g = (jnp.dot(Af, oh, preferred_element_type=jnp.float32)
             + jnp.dot(Whh, h, preferred_element_type=jnp.float32) + bf)
        i_g = jax.nn.sigmoid(g[0:LSTM])
        f_g = jax.nn.sigmoid(g[LSTM:2 * LSTM])
        gg = jnp.tanh(g[2 * LSTM:3 * LSTM])
        o_g = jax.nn.sigmoid(g[3 * LSTM:4 * LSTM])
        c = f_g * c + i_g * gg
        h = o_g * jnp.tanh(c)

    Ar = jnp.dot(wihr_ref[...], embT, preferred_element_type=jnp.float32)
    gr = jnp.dot(Ar, onehot(tok[SEQ_LEN - 1]),
                 preferred_element_type=jnp.float32) + br_ref[...]
    c_r = jax.nn.sigmoid(gr[0:LSTM]) * jnp.tanh(gr[2 * LSTM:3 * LSTM])
    h_r = jax.nn.sigmoid(gr[3 * LSTM:4 * LSTM]) * jnp.tanh(c_r)

    hcat = jnp.concatenate([h, h_r], axis=0)
    v = jax.nn.relu(jnp.dot(l1w_ref[...], hcat,
                            preferred_element_type=jnp.float32) + l1b_ref[...])
    s = jax.nn.relu(jnp.dot(l2w_ref[...], v,
                            preferred_element_type=jnp.float32) + l2b_ref[...])
    out_ref[0, 0, :] = s[0]


def _edge_scores(tokens2, embed, Wih_f, Whh_f, bih_f, bhh_f,
                 Wih_r, bih_r, bhh_r, lin1_W, lin1_b, lin2_W, lin2_b):
    tokT = tokens2.T.astype(jnp.int32)
    embT = embed.at[0].set(0.0).T
    bf = (bih_f + bhh_f)[:, None]
    br = (bih_r + bhh_r)[:, None]
    full = lambda shape: pl.BlockSpec(shape, lambda i: (0,) * len(shape))
    out = pl.pallas_call(
        _edge_score_body,
        grid=(NBLK,),
        in_specs=[
            pl.BlockSpec((SEQ_LEN, EB), lambda i: (0, i)),
            full((EMB, VOCAB)),
            full((H4, EMB)),
            full((H4, LSTM)),
            full((H4, 1)),
            full((H4, EMB)),
            full((H4, 1)),
            full((32, 2 * LSTM)),
            full((32, 1)),
            full((1, 32)),
            full((1, 1)),
        ],
        out_specs=pl.BlockSpec((1, 1, EB), lambda i: (i, 0, 0)),
        out_shape=jax.ShapeDtypeStruct((NBLK, 1, EB), jnp.float32),
    )(tokT, embT, Wih_f, Whh_f, bf, Wih_r, br,
      lin1_W, lin1_b[:, None], lin2_W, lin2_b[:, None])
    return out.reshape(E2)


def _overwrite(x, row_idx, col_idx, val):
    key = row_idx * 64 + col_idx
    w = jnp.full((N_NODES * 64,), -1, jnp.int32)
    w = w.at[key].max(jnp.arange(N_EDGES, dtype=jnp.int32))
    w2 = w.reshape(N_NODES, 64)[:, :TID]
    cell = jnp.where(w2 >= 0, val[jnp.clip(w2, 0)], x[:, REGEX_IDX:REGEX_IDX + TID])
    return jnp.concatenate([x[:, :REGEX_IDX], cell], axis=1)


def _gatv2(x, src, dst, Wl, bl, Wr, br, att, bias):
    N = x.shape[0]
    xl = x @ Wl.T + bl
    xr = x @ Wr.T + br
    e = jax.nn.leaky_relu(xl[src] + xr[dst], 0.2)
    logit = e @ att
    m = jax.ops.segment_max(logit, dst, num_segments=N)
    m = jnp.where(jnp.isfinite(m), m, 0.0)
    ex = jnp.exp(logit - m[dst])
    den = jax.ops.segment_sum(ex, dst, num_segments=N)
    alpha = ex / (den[dst] + 1e-16)
    out = jax.ops.segment_sum(alpha[:, None] * xl[src], dst, num_segments=N)
    return out + bias


def kernel(fwd_x, fwd_edge_index, fwd_edge_attr, bwd_x, bwd_edge_index,
           bwd_edge_attr, embed, Wih_f, Whh_f, bih_f, bhh_f, Wih_r, Whh_r,
           bih_r, bhh_r, lin1_W, lin1_b, lin2_W, lin2_b, fgat_Wl, fgat_bl,
           fgat_Wr, fgat_br, fgat_att, fgat_bias, bgat_Wl, bgat_bl, bgat_Wr,
           bgat_br, bgat_att, bgat_bias):
    tokens2 = jnp.concatenate([fwd_edge_attr, bwd_edge_attr], axis=0)
    vals = _edge_scores(tokens2, embed, Wih_f, Whh_f, bih_f, bhh_f,
                        Wih_r, bih_r, bhh_r, lin1_W, lin1_b, lin2_W, lin2_b)
    f_val, b_val = vals[:N_EDGES], vals[N_EDGES:]

    f_src, f_dst = fwd_edge_index[0], fwd_edge_index[1]
    b_src, b_dst = bwd_edge_index[0], bwd_edge_index[1]
    f_tidn = jnp.argmax(fwd_x[:, :TID], axis=-1).astype(jnp.int32)
    b_tidn = jnp.argmax(bwd_x[:, :TID], axis=-1).astype(jnp.int32)

    fx = _overwrite(fwd_x, f_src, f_tidn[f_dst], f_val)
    bx = _overwrite(bwd_x, b_dst, b_tidn[b_dst], b_val)

    fx = jax.nn.relu(_gatv2(fx, f_src, f_dst, fgat_Wl, fgat_bl, fgat_Wr,
                            fgat_br, fgat_att, fgat_bias) + fx)
    bx = jax.nn.relu(_gatv2(bx, b_src, b_dst, bgat_Wl, bgat_bl, bgat_Wr,
                            bgat_br, bgat_att, bgat_bias) + bx)
    return jnp.concatenate([fx, bx], axis=-1)


# speedup vs baseline: 43.6590x; 4.6186x over previous
import functools

import jax
import jax.numpy as jnp
from jax import lax
from jax.experimental import pallas as pl
from jax.experimental.pallas import tpu as pltpu
from jax.experimental.pallas import tpu_sc as plsc

MAX_STATES = 50
TID = MAX_STATES + 3
REGEX_IDX = TID + 2 + TID + TID
HID = REGEX_IDX + TID
N_NODES = 10000
N_EDGES = 160000
SEQ_LEN = 8
VOCAB = 100
EMB = 32
LSTM = 64
H4 = 4 * LSTM

EB = 2560
E2 = 2 * N_EDGES
NBLK = E2 // EB

NKEY = N_NODES * 64
KW = NKEY // 16
ECH = 2000

NB = 1000
DP = 224
DH = 112
EPW = N_EDGES // 16
KCH = 80
NPS = N_NODES // 16
NPAD = 10240

_SC_PARAMS = pltpu.CompilerParams(needs_layout_passes=False,
                                  use_tc_tiling_on_sc=False)


def _edge_score_body(tok_ref, embT_ref, wihf_ref, whhf_ref, bf_ref,
                     wihr_ref, br_ref, l1w_ref, l1b_ref, l2w_ref, l2b_ref,
                     out_ref):
    tok = tok_ref[...]
    embT = embT_ref[...]
    Af = jnp.dot(wihf_ref[...], embT, preferred_element_type=jnp.float32)
    Whh = whhf_ref[...]
    bf = bf_ref[...]

    def onehot(row):
        i = jax.lax.broadcasted_iota(jnp.int32, (VOCAB, EB), 0)
        return (i == row[None, :]).astype(jnp.float32)

    h = jnp.zeros((LSTM, EB), jnp.float32)
    c = jnp.zeros((LSTM, EB), jnp.float32)
    for t in range(SEQ_LEN):
        oh = onehot(tok[t])
        g = (jnp.dot(Af, oh, preferred_element_type=jnp.float32)
             + jnp.dot(Whh, h, preferred_element_type=jnp.float32) + bf)
        i_g = jax.nn.sigmoid(g[0:LSTM])
        f_g = jax.nn.sigmoid(g[LSTM:2 * LSTM])
        gg = jnp.tanh(g[2 * LSTM:3 * LSTM])
        o_g = jax.nn.sigmoid(g[3 * LSTM:4 * LSTM])
        c = f_g * c + i_g * gg
        h = o_g * jnp.tanh(c)

    Ar = jnp.dot(wihr_ref[...], embT, preferred_element_type=jnp.float32)
    gr = jnp.dot(Ar, onehot(tok[SEQ_LEN - 1]),
                 preferred_element_type=jnp.float32) + br_ref[...]
    c_r = jax.nn.sigmoid(gr[0:LSTM]) * jnp.tanh(gr[2 * LSTM:3 * LSTM])
    h_r = jax.nn.sigmoid(gr[3 * LSTM:4 * LSTM]) * jnp.tanh(c_r)

    hcat = jnp.concatenate([h, h_r], axis=0)
    v = jax.nn.relu(jnp.dot(l1w_ref[...], hcat,
                            preferred_element_type=jnp.float32) + l1b_ref[...])
    s = jax.nn.relu(jnp.dot(l2w_ref[...], v,
                            preferred_element_type=jnp.float32) + l2b_ref[...])
    out_ref[0, 0, :] = s[0]


def _edge_scores(tokens2, embed, Wih_f, Whh_f, bih_f, bhh_f,
                 Wih_r, bih_r, bhh_r, lin1_W, lin1_b, lin2_W, lin2_b):
    tokT = tokens2.T.astype(jnp.int32)
    embT = embed.at[0].set(0.0).T
    bf = (bih_f + bhh_f)[:, None]
    br = (bih_r + bhh_r)[:, None]
    full = lambda shape: pl.BlockSpec(shape, lambda i: (0,) * len(shape))
    out = pl.pallas_call(
        _edge_score_body,
        grid=(NBLK,),
        in_specs=[
            pl.BlockSpec((SEQ_LEN, EB), lambda i: (0, i)),
            full((EMB, VOCAB)),
            full((H4, EMB)),
            full((H4, LSTM)),
            full((H4, 1)),
            full((H4, EMB)),
            full((H4, 1)),
            full((32, 2 * LSTM)),
            full((32, 1)),
            full((1, 32)),
            full((1, 1)),
        ],
        out_specs=pl.BlockSpec((1, 1, EB), lambda i: (i, 0, 0)),
        out_shape=jax.ShapeDtypeStruct((NBLK, 1, EB), jnp.float32),
    )(tokT, embT, Wih_f, Whh_f, bf, Wih_r, br,
      lin1_W, lin1_b[:, None], lin2_W, lin2_b[:, None])
    return out.reshape(E2)


def _tid_body(x_ref, out_ref):
    t = x_ref[0][:, :TID]
    m = jnp.max(t, axis=1, keepdims=True)
    iota = jax.lax.broadcasted_iota(jnp.int32, (NB, TID), 1)
    idx = jnp.min(jnp.where(t == m, iota, TID), axis=1, keepdims=True)
    out_ref[0] = jnp.broadcast_to(idx, (NB, 8))


def _tid_argmax(x2):
    out = pl.pallas_call(
        _tid_body,
        grid=(2, N_NODES // NB),
        in_specs=[pl.BlockSpec((1, NB, HID), lambda s, i: (s, i, 0))],
        out_specs=pl.BlockSpec((1, NB, 8), lambda s, i: (s, i, 0)),
        out_shape=jax.ShapeDtypeStruct((2, N_NODES, 8), jnp.int32),
    )(x2)
    return out[:, :, 0].reshape(2 * N_NODES)


def _k1_body(rows_hbm, dst_hbm, vals_hbm, tid_hbm, p_hbm, v_hbm,
             tidv, rbuf, dbuf, vbuf, pv, vv, tmp):
    core = lax.axis_index("c")
    sub = lax.axis_index("s")
    base = sub * KW
    e0 = core * N_EDGES
    pltpu.sync_copy(tid_hbm.at[pl.ds(core * N_NODES, N_NODES)], tidv)

    def zero(i, _):
        pv[pl.ds(i * 16, 16)] = jnp.zeros((16,), jnp.int32)
        return 0
    lax.fori_loop(0, KW // 16, zero, 0)

    lanes = lax.iota(jnp.int32, 16)
    tmp[pl.ds(16, 16)] = jnp.full((16,), -1, jnp.int32)
    ones = jnp.ones((16,), jnp.int32)

    def chunk(i, _):
        pltpu.sync_copy(rows_hbm.at[pl.ds(e0 + i * ECH, ECH)], rbuf)
        pltpu.sync_copy(dst_hbm.at[pl.ds(e0 + i * ECH, ECH)], dbuf)
        pltpu.sync_copy(vals_hbm.at[pl.ds(e0 + i * ECH, ECH)], vbuf)

        def step(j, _):
            r = rbuf[pl.ds(j * 16, 16)]
            d = dbuf[pl.ds(j * 16, 16)]
            v = vbuf[pl.ds(j * 16, 16)]
            t = plsc.load_gather(tidv, [d])
            key = r * 64 + t
            skey = key * 16 + lanes
            ks, vs = plsc.sort_key_val(skey, v)
            tmp[pl.ds(0, 16)] = ks
            nx = plsc.load_gather(tmp, [lanes + 1])
            kq = lax.shift_right_logical(ks, 4)
            nq = lax.shift_right_logical(nx, 4)
            msk = (kq != nq) & (kq >= base) & (kq < base + KW)
            loc = jnp.clip(kq - base, 0, KW - 1)
            plsc.store_scatter(vv, [loc], vs, mask=msk)
            plsc.store_scatter(pv, [loc], ones, mask=msk)
            return 0
        lax.fori_loop(0, ECH // 16, step, 0)
        return 0
    lax.fori_loop(0, N_EDGES // ECH, chunk, 0)

    pltpu.sync_copy(pv, p_hbm.at[pl.ds(core * NKEY + base, KW)])
    pltpu.sync_copy(vv, v_hbm.at[pl.ds(core * NKEY + base, KW)])


def _k1_overwrite_winner(rows2, dst2, vals2, tid2):
    mesh = plsc.VectorSubcoreMesh(core_axis_name="c", subcore_axis_name="s")
    f = pl.kernel(
        _k1_body,
        mesh=mesh,
        compiler_params=_SC_PARAMS,
        out_type=[
            jax.ShapeDtypeStruct((2 * NKEY,), jnp.int32),
            jax.ShapeDtypeStruct((2 * NKEY,), jnp.float32),
        ],
        scratch_types=[
            pltpu.VMEM((N_NODES,), jnp.int32),
            pltpu.VMEM((ECH,), jnp.int32),
            pltpu.VMEM((ECH,), jnp.int32),
            pltpu.VMEM((ECH,), jnp.float32),
            pltpu.VMEM((KW,), jnp.int32),
            pltpu.VMEM((KW,), jnp.float32),
            pltpu.VMEM((32,), jnp.int32),
        ],
    )
    return f(rows2, dst2, vals2, tid2)


def _compose_body(x_ref, p_ref, v_ref, wlT_ref, bl_ref, wrT_ref, br_ref,
                  fx_ref, xl_ref, xr_ref, xlL_ref, xlH_ref):
    x = x_ref[0]
    p = p_ref[0][:, :TID]
    v = v_ref[0][:, :TID]
    cell = jnp.where(p > 0, v, x[:, REGEX_IDX:])
    fxb = jnp.concatenate([x[:, :REGEX_IDX], cell], axis=1)
    fx_ref[0] = fxb
    xl = jnp.dot(fxb, wlT_ref[0], preferred_element_type=jnp.float32) + bl_ref[0]
    xr = jnp.dot(fxb, wrT_ref[0], preferred_element_type=jnp.float32) + br_ref[0]
    one = jnp.ones((NB, 1), jnp.float32)
    z9 = jnp.zeros((NB, 9), jnp.float32)
    xl_ref[0] = jnp.concatenate([xl, one, z9], axis=1)
    xr_ref[0] = jnp.concatenate([xr, jnp.zeros((NB, 10), jnp.float32)], axis=1)
    xlL_ref[0] = xl[:, :DH]
    xlH_ref[0] = jnp.concatenate([xl[:, DH:], one, z9], axis=1)


def _compose_transform(x2, P, V, wlT2, bl2, wrT2, br2):
    p3 = P.reshape(2, N_NODES, 64)
    v3 = V.reshape(2, N_NODES, 64)
    sds = jax.ShapeDtypeStruct
    return pl.pallas_call(
        _compose_body,
        grid=(2, N_NODES // NB),
        in_specs=[
            pl.BlockSpec((1, NB, HID), lambda s, i: (s, i, 0)),
            pl.BlockSpec((1, NB, 64), lambda s, i: (s, i, 0)),
            pl.BlockSpec((1, NB, 64), lambda s, i: (s, i, 0)),
            pl.BlockSpec((1, HID, HID), lambda s, i: (s, 0, 0)),
            pl.BlockSpec((1, 1, HID), lambda s, i: (s, 0, 0)),
            pl.BlockSpec((1, HID, HID), lambda s, i: (s, 0, 0)),
            pl.BlockSpec((1, 1, HID), lambda s, i: (s, 0, 0)),
        ],
        out_specs=[
            pl.BlockSpec((1, NB, HID), lambda s, i: (s, i, 0)),
            pl.BlockSpec((1, NB, DP), lambda s, i: (s, i, 0)),
            pl.BlockSpec((1, NB, DP), lambda s, i: (s, i, 0)),
            pl.BlockSpec((1, NB, DH), lambda s, i: (s, i, 0)),
            pl.BlockSpec((1, NB, DH), lambda s, i: (s, i, 0)),
        ],
        out_shape=[
            sds((2, N_NODES, HID), jnp.float32),
            sds((2, N_NODES, DP), jnp.float32),
            sds((2, N_NODES, DP), jnp.float32),
            sds((2, N_NODES, DH), jnp.float32),
            sds((2, N_NODES, DH), jnp.float32),
        ],
    )(x2, p3, v3, wlT2, bl2, wrT2, br2)


def _k3_body(src_hbm, dst_hbm, att_hbm, xl_hbm, xr_hbm, logit_hbm, m_hbm,
             sidx, didx, d2idx, zl, zr, lbuf, m_local, mrg, mout, attv, sem,
             mshared):
    core = lax.axis_index("c")
    sub = lax.axis_index("s")
    e0 = core * N_EDGES + sub * EPW
    pltpu.sync_copy(att_hbm.at[pl.ds(core * DP, DP)], attv)
    att_regs = [attv[pl.ds(j * 16, 16)] for j in range(DP // 16)]

    def initm(i, _):
        m_local[pl.ds(i * 16, 16)] = jnp.full((16,), -3e38, jnp.float32)
        return 0
    lax.fori_loop(0, N_NODES // 16, initm, 0)

    lanes = lax.iota(jnp.int32, 16)

    def chunk(ch, _):
        eb = e0 + ch * KCH
        pltpu.sync_copy(src_hbm.at[pl.ds(eb, KCH)], sidx)
        pltpu.sync_copy(dst_hbm.at[pl.ds(eb, KCH)], didx)
        for j in range(KCH // 16):
            sl = pl.ds(j * 16, 16)
            sidx[sl] = sidx[sl] + core * N_NODES
            d2idx[sl] = didx[sl] + core * N_NODES
        pltpu.async_copy(xl_hbm.at[sidx], zl, sem).wait()
        pltpu.async_copy(xr_hbm.at[d2idx], zr, sem).wait()

        def edge(e, _):
            acc = jnp.zeros((16,), jnp.float32)
            for j in range(DP // 16):
                z = zl[e, pl.ds(j * 16, 16)] + zr[e, pl.ds(j * 16, 16)]
                lk = 0.6 * z + 0.4 * jnp.abs(z)
                acc = acc + lk * att_regs[j]
            lg = jnp.sum(acc)
            plsc.store_scatter(lbuf, [jnp.full((16,), 0, jnp.int32) + e],
                               jnp.zeros((16,), jnp.float32) + lg,
                               mask=lanes == 0)
            return 0
        lax.fori_loop(0, KCH, edge, 0)
        pltpu.sync_copy(lbuf, logit_hbm.at[pl.ds(eb, KCH)])

        for j in range(KCH // 16):
            sl = pl.ds(j * 16, 16)
            dv = didx[sl]
            lv = lbuf[sl]

            def mcond(st):
                return st[1] > 0

            def mbody(st):
                nv = st[0]
                cur = plsc.load_gather(m_local, [dv])
                act = nv > cur
                plsc.store_scatter(m_local, [dv], nv, mask=act)
                cur2 = plsc.load_gather(m_local, [dv])
                cnt = jnp.sum((nv > cur2).astype(jnp.int32))
                return (nv, cnt)
            lax.while_loop(mcond, mbody, (lv, jnp.int32(1)))
        return 0
    lax.fori_loop(0, EPW // KCH, chunk, 0)

    pltpu.sync_copy(m_local, mshared.at[pl.ds(sub * NPAD, N_NODES)])
    plsc.subcore_barrier()
    for r in range(16):
        pltpu.sync_copy(mshared.at[pl.ds(r * NPAD + sub * 640, 640)],
                        mrg.at[pl.ds(r * 640, 640)])
    for j in range(40):
        a = mrg[pl.ds(j * 16, 16)]
        for rr in range(1, 16):
            a = jnp.maximum(a, mrg[pl.ds(rr * 640 + j * 16, 16)])
        a = jnp.where(a < -1e38, 0.0, a)
        mout[pl.ds(j * 16, 16)] = a
    pltpu.sync_copy(mout, m_hbm.at[pl.ds(core * NPAD + sub * 640, 640)])


def _k3_logits(src2, dst2, att2, xl4, xr4):
    mesh = plsc.VectorSubcoreMesh(core_axis_name="c", subcore_axis_name="s")
    f = pl.kernel(
        _k3_body,
        mesh=mesh,
        compiler_params=_SC_PARAMS,
        out_type=[
            jax.ShapeDtypeStruct((E2,), jnp.float32),
            jax.ShapeDtypeStruct((2 * NPAD,), jnp.float32),
        ],
        scratch_types=[
            pltpu.VMEM((KCH,), jnp.int32),
            pltpu.VMEM((KCH,), jnp.int32),
            pltpu.VMEM((KCH,), jnp.int32),
            pltpu.VMEM((KCH, DP), jnp.float32),
            pltpu.VMEM((KCH, DP), jnp.float32),
            pltpu.VMEM((KCH,), jnp.float32),
            pltpu.VMEM((N_NODES,), jnp.float32),
            pltpu.VMEM((16 * 640,), jnp.float32),
            pltpu.VMEM((640,), jnp.float32),
            pltpu.VMEM((DP,), jnp.float32),
            pltpu.SemaphoreType.DMA,
            pltpu.VMEM_SHARED((16 * NPAD,), jnp.float32),
        ],
    )
    return f(src2, dst2, att2, xl4, xr4)


def _k5_body(src_hbm, dst_hbm, logit_hbm, xlL_hbm, xlH_hbm, m_hbm, out_hbm,
             sidx, didx, lbuf, exbuf, rows, m_local, zbuf, sem, acc):
    core = lax.axis_index("c")
    sub = lax.axis_index("s")
    e0 = core * N_EDGES + sub * EPW
    pltpu.sync_copy(m_hbm.at[pl.ds(core * NPAD, N_NODES)], m_local)

    def zinit(i, _):
        j = i % 7
        r = i // 7
        zbuf[r, pl.ds(j * 16, 16)] = jnp.zeros((16,), jnp.float32)
        return 0
    lax.fori_loop(0, 25 * 7, zinit, 0)

    for half in range(2):
        tab = xlL_hbm if half == 0 else xlH_hbm

        def zacc(i, _):
            pltpu.sync_copy(zbuf, acc.at[pl.ds(sub * NPS + i * 25, 25)])
            return 0
        lax.fori_loop(0, NPS // 25, zacc, 0)
        plsc.subcore_barrier()

        def chunk(ch, _):
            eb = e0 + ch * KCH
            pltpu.sync_copy(src_hbm.at[pl.ds(eb, KCH)], sidx)
            pltpu.sync_copy(dst_hbm.at[pl.ds(eb, KCH)], didx)
            pltpu.sync_copy(logit_hbm.at[pl.ds(eb, KCH)], lbuf)
            for j in range(KCH // 16):
                sl = pl.ds(j * 16, 16)
                dv = didx[sl]
                mg = plsc.load_gather(m_local, [dv])
                exbuf[sl] = jnp.exp(lbuf[sl] - mg)
                sidx[sl] = sidx[sl] + core * N_NODES
            pltpu.async_copy(tab.at[sidx], rows, sem).wait()

            def edge(e, _):
                w = exbuf[pl.ds(e, 16)][0]
                for j in range(DH // 16):
                    sl = pl.ds(j * 16, 16)
                    rows[e, sl] = rows[e, sl] * w
                return 0
            lax.fori_loop(0, KCH, edge, 0)
            pltpu.sync_copy(rows, acc.at[didx], add=True)
            return 0
        lax.fori_loop(0, EPW // KCH, chunk, 0)
        plsc.subcore_barrier()
        pltpu.sync_copy(
            acc.at[pl.ds(sub * NPS, NPS)],
            out_hbm.at[pl.ds(half * 2 * N_NODES + core * N_NODES
                             + sub * NPS, NPS)])


def _k5_accumulate(src2, dst2, logits, xlL4, xlH4, m2):
    mesh = plsc.VectorSubcoreMesh(core_axis_name="c", subcore_axis_name="s")
    f = pl.kernel(
        _k5_body,
        mesh=mesh,
        compiler_params=_SC_PARAMS,
        out_type=jax.ShapeDtypeStruct((4 * N_NODES, DH), jnp.float32),
        scratch_types=[
            pltpu.VMEM((KCH,), jnp.int32),
            pltpu.VMEM((KCH,), jnp.int32),
            pltpu.VMEM((KCH,), jnp.float32),
            pltpu.VMEM((KCH + 16,), jnp.float32),
            pltpu.VMEM((KCH, DH), jnp.float32),
            pltpu.VMEM((N_NODES,), jnp.float32),
            pltpu.VMEM((25, DH), jnp.float32),
            pltpu.SemaphoreType.DMA,
            pltpu.VMEM_SHARED((N_NODES, DH), jnp.float32),
        ],
    )
    return f(src2, dst2, logits, xlL4, xlH4, m2)


def _final_body(accL_ref, accH_ref, fx_ref, bias_ref, out_ref):
    accL = accL_ref[0]
    accH = accH_ref[0]
    den = accH[:, 102:103] + 1e-16
    num = jnp.concatenate([accL, accH[:, :102]], axis=1)
    out_ref[0] = jax.nn.relu(num / den + bias_ref[0] + fx_ref[0])


def _final(accL, accH, fx, bias2):
    out = pl.pallas_call(
        _final_body,
        grid=(2, N_NODES // NB),
        in_specs=[
            pl.BlockSpec((1, NB, DH), lambda s, i: (s, i, 0)),
            pl.BlockSpec((1, NB, DH), lambda s, i: (s, i, 0)),
            pl.BlockSpec((1, NB, HID), lambda s, i: (s, i, 0)),
            pl.BlockSpec((1, 1, HID), lambda s, i: (s, 0, 0)),
        ],
        out_specs=pl.BlockSpec((1, NB, HID), lambda s, i: (s, i, 0)),
        out_shape=jax.ShapeDtypeStruct((2, N_NODES, HID), jnp.float32),
    )(accL, accH, fx, bias2)
    return out


def kernel(fwd_x, fwd_edge_index, fwd_edge_attr, bwd_x, bwd_edge_index,
           bwd_edge_attr, embed, Wih_f, Whh_f, bih_f, bhh_f, Wih_r, Whh_r,
           bih_r, bhh_r, lin1_W, lin1_b, lin2_W, lin2_b, fgat_Wl, fgat_bl,
           fgat_Wr, fgat_br, fgat_att, fgat_bias, bgat_Wl, bgat_bl, bgat_Wr,
           bgat_br, bgat_att, bgat_bias):
    tokens2 = jnp.concatenate([fwd_edge_attr, bwd_edge_attr], axis=0)
    vals2 = _edge_scores(tokens2, embed, Wih_f, Whh_f, bih_f, bhh_f,
                         Wih_r, bih_r, bhh_r, lin1_W, lin1_b, lin2_W, lin2_b)

    f_src, f_dst = fwd_edge_index[0], fwd_edge_index[1]
    b_src, b_dst = bwd_edge_index[0], bwd_edge_index[1]
    x2 = jnp.stack([fwd_x, bwd_x])
    tid2 = _tid_argmax(x2)

    rows2 = jnp.concatenate([f_src, b_dst]).astype(jnp.int32)
    src2 = jnp.concatenate([f_src, b_src]).astype(jnp.int32)
    dst2 = jnp.concatenate([f_dst, b_dst]).astype(jnp.int32)
    P, V = _k1_overwrite_winner(rows2, dst2, vals2, tid2)

    wlT2 = jnp.stack([fgat_Wl.T, bgat_Wl.T])
    wrT2 = jnp.stack([fgat_Wr.T, bgat_Wr.T])
    bl2 = jnp.stack([fgat_bl, bgat_bl])[:, None, :]
    br2 = jnp.stack([fgat_br, bgat_br])[:, None, :]
    fx2, xl2, xr2, xlL2, xlH2 = _compose_transform(x2, P, V, wlT2, bl2,
                                                   wrT2, br2)

    att2 = jnp.concatenate([
        jnp.pad(fgat_att, (0, DP - HID)), jnp.pad(bgat_att, (0, DP - HID))])
    xl4 = xl2.reshape(2 * N_NODES, DP)
    xr4 = xr2.reshape(2 * N_NODES, DP)
    logits, m2 = _k3_logits(src2, dst2, att2, xl4, xr4)

    xlL4 = xlL2.reshape(2 * N_NODES, DH)
    xlH4 = xlH2.reshape(2 * N_NODES, DH)
    accs = _k5_accumulate(src2, dst2, logits, xlL4, xlH4, m2)
    accL = accs[:2 * N_NODES].reshape(2, N_NODES, DH)
    accH = accs[2 * N_NODES:].reshape(2, N_NODES, DH)

    bias2 = jnp.stack([fgat_bias, bgat_bias])[:, None, :]
    out2 = _final(accL, accH, fx2, bias2)
    return jnp.concatenate([out2[0], out2[1]], axis=1)


# speedup vs baseline: 51.5380x; 1.1805x over previous
import functools

import jax
import jax.numpy as jnp
from jax import lax
from jax.experimental import pallas as pl
from jax.experimental.pallas import tpu as pltpu
from jax.experimental.pallas import tpu_sc as plsc

MAX_STATES = 50
TID = MAX_STATES + 3
REGEX_IDX = TID + 2 + TID + TID
HID = REGEX_IDX + TID
N_NODES = 10000
N_EDGES = 160000
SEQ_LEN = 8
VOCAB = 100
EMB = 32
LSTM = 64
H4 = 4 * LSTM

EB = 2560
E2 = 2 * N_EDGES
NBLK = E2 // EB

NKEY = N_NODES * 64
KW = NKEY // 16
ECH = 2000

NB = 1000
DP = 224
DH = 112
EPW = N_EDGES // 16
KCH = 80
NPS = N_NODES // 16
NPAD = 10240

_SC_PARAMS = pltpu.CompilerParams(needs_layout_passes=False,
                                  use_tc_tiling_on_sc=False)


def _edge_score_body(tok_ref, embT_ref, wihf_ref, whhf_ref, bf_ref,
                     wihr_ref, br_ref, l1w_ref, l1b_ref, l2w_ref, l2b_ref,
                     out_ref):
    tok = tok_ref[...]
    embT = embT_ref[...]
    Af = jnp.dot(wihf_ref[...], embT, preferred_element_type=jnp.float32)
    Whh = whhf_ref[...]
    bf = bf_ref[...]

    def onehot(row):
        i = jax.lax.broadcasted_iota(jnp.int32, (VOCAB, EB), 0)
        return (i == row[None, :]).astype(jnp.float32)

    sig = jax.nn.sigmoid
    tnh = jnp.tanh

    h = jnp.zeros((LSTM, EB), jnp.float32)
    c = jnp.zeros((LSTM, EB), jnp.float32)
    for t in range(SEQ_LEN):
        oh = onehot(tok[t])
        g = (jnp.dot(Af, oh, preferred_element_type=jnp.float32)
             + jnp.dot(Whh, h, preferred_element_type=jnp.float32) + bf)
        i_g = sig(g[0:LSTM])
        f_g = sig(g[LSTM:2 * LSTM])
        gg = tnh(g[2 * LSTM:3 * LSTM])
        o_g = sig(g[3 * LSTM:4 * LSTM])
        c = f_g * c + i_g * gg
        h = o_g * tnh(c)

    Ar = jnp.dot(wihr_ref[...], embT, preferred_element_type=jnp.float32)
    gr = jnp.dot(Ar, onehot(tok[SEQ_LEN - 1]),
                 preferred_element_type=jnp.float32) + br_ref[...]
    c_r = sig(gr[0:LSTM]) * tnh(gr[2 * LSTM:3 * LSTM])
    h_r = sig(gr[3 * LSTM:4 * LSTM]) * tnh(c_r)

    hcat = jnp.concatenate([h, h_r], axis=0)
    v = jax.nn.relu(jnp.dot(l1w_ref[...], hcat,
                            preferred_element_type=jnp.float32) + l1b_ref[...])
    s = jax.nn.relu(jnp.dot(l2w_ref[...], v,
                            preferred_element_type=jnp.float32) + l2b_ref[...])
    out_ref[0, 0, :] = s[0]


def _edge_scores(tokens2, embed, Wih_f, Whh_f, bih_f, bhh_f,
                 Wih_r, bih_r, bhh_r, lin1_W, lin1_b, lin2_W, lin2_b):
    tokT = tokens2.T.astype(jnp.int32)
    embT = embed.at[0].set(0.0).T
    bf = (bih_f + bhh_f)[:, None]
    br = (bih_r + bhh_r)[:, None]
    full = lambda shape: pl.BlockSpec(shape, lambda i: (0,) * len(shape))
    out = pl.pallas_call(
        _edge_score_body,
        grid=(NBLK,),
        in_specs=[
            pl.BlockSpec((SEQ_LEN, EB), lambda i: (0, i)),
            full((EMB, VOCAB)),
            full((H4, EMB)),
            full((H4, LSTM)),
            full((H4, 1)),
            full((H4, EMB)),
            full((H4, 1)),
            full((32, 2 * LSTM)),
            full((32, 1)),
            full((1, 32)),
            full((1, 1)),
        ],
        out_specs=pl.BlockSpec((1, 1, EB), lambda i: (i, 0, 0)),
        out_shape=jax.ShapeDtypeStruct((NBLK, 1, EB), jnp.float32),
    )(tokT, embT, Wih_f, Whh_f, bf, Wih_r, br,
      lin1_W, lin1_b[:, None], lin2_W, lin2_b[:, None])
    return out.reshape(E2)


def _tid_body(x_ref, out_ref):
    t = x_ref[0][:, :TID]
    m = jnp.max(t, axis=1, keepdims=True)
    iota = jax.lax.broadcasted_iota(jnp.int32, (NB, TID), 1)
    idx = jnp.min(jnp.where(t == m, iota, TID), axis=1, keepdims=True)
    out_ref[0] = jnp.broadcast_to(idx, (NB, 8))


def _tid_argmax(x2):
    out = pl.pallas_call(
        _tid_body,
        grid=(2, N_NODES // NB),
        in_specs=[pl.BlockSpec((1, NB, HID), lambda s, i: (s, i, 0))],
        out_specs=pl.BlockSpec((1, NB, 8), lambda s, i: (s, i, 0)),
        out_shape=jax.ShapeDtypeStruct((2, N_NODES, 8), jnp.int32),
    )(x2)
    return out[:, :, 0].reshape(2 * N_NODES)


def _k1_body(rows_hbm, dst_hbm, vals_hbm, tid_hbm, p_hbm, v_hbm,
             tidv, rbuf, dbuf, vbuf, pv, vv, tmp):
    core = lax.axis_index("c")
    sub = lax.axis_index("s")
    base = sub * KW
    e0 = core * N_EDGES
    pltpu.sync_copy(tid_hbm.at[pl.ds(core * N_NODES, N_NODES)], tidv)

    def zero(i, _):
        pv[pl.ds(i * 16, 16)] = jnp.zeros((16,), jnp.int32)
        return 0
    lax.fori_loop(0, KW // 16, zero, 0)

    lanes = lax.iota(jnp.int32, 16)
    tmp[pl.ds(16, 16)] = jnp.full((16,), -1, jnp.int32)
    ones = jnp.ones((16,), jnp.int32)

    def chunk(i, _):
        pltpu.sync_copy(rows_hbm.at[pl.ds(e0 + i * ECH, ECH)], rbuf)
        pltpu.sync_copy(dst_hbm.at[pl.ds(e0 + i * ECH, ECH)], dbuf)
        pltpu.sync_copy(vals_hbm.at[pl.ds(e0 + i * ECH, ECH)], vbuf)

        def step(j, _):
            r = rbuf[pl.ds(j * 16, 16)]
            d = dbuf[pl.ds(j * 16, 16)]
            v = vbuf[pl.ds(j * 16, 16)]
            t = plsc.load_gather(tidv, [d])
            key = r * 64 + t
            skey = key * 16 + lanes
            ks, vs = plsc.sort_key_val(skey, v)
            tmp[pl.ds(0, 16)] = ks
            nx = plsc.load_gather(tmp, [lanes + 1])
            kq = lax.shift_right_logical(ks, 4)
            nq = lax.shift_right_logical(nx, 4)
            msk = (kq != nq) & (kq >= base) & (kq < base + KW)
            loc = jnp.clip(kq - base, 0, KW - 1)
            plsc.store_scatter(vv, [loc], vs, mask=msk)
            plsc.store_scatter(pv, [loc], ones, mask=msk)
            return 0
        lax.fori_loop(0, ECH // 16, step, 0)
        return 0
    lax.fori_loop(0, N_EDGES // ECH, chunk, 0)

    pltpu.sync_copy(pv, p_hbm.at[pl.ds(core * NKEY + base, KW)])
    pltpu.sync_copy(vv, v_hbm.at[pl.ds(core * NKEY + base, KW)])


def _k1_overwrite_winner(rows2, dst2, vals2, tid2):
    mesh = plsc.VectorSubcoreMesh(core_axis_name="c", subcore_axis_name="s")
    f = pl.kernel(
        _k1_body,
        mesh=mesh,
        compiler_params=_SC_PARAMS,
        out_type=[
            jax.ShapeDtypeStruct((2 * NKEY,), jnp.int32),
            jax.ShapeDtypeStruct((2 * NKEY,), jnp.float32),
        ],
        scratch_types=[
            pltpu.VMEM((N_NODES,), jnp.int32),
            pltpu.VMEM((ECH,), jnp.int32),
            pltpu.VMEM((ECH,), jnp.int32),
            pltpu.VMEM((ECH,), jnp.float32),
            pltpu.VMEM((KW,), jnp.int32),
            pltpu.VMEM((KW,), jnp.float32),
            pltpu.VMEM((32,), jnp.int32),
        ],
    )
    return f(rows2, dst2, vals2, tid2)


def _compose_body(x_ref, p_ref, v_ref, wlT_ref, bl_ref, wrT_ref, br_ref,
                  fx_ref, xl_ref, xr_ref, xlL_ref, xlH_ref):
    x = x_ref[0]
    p = p_ref[0][:, :TID]
    v = v_ref[0][:, :TID]
    cell = jnp.where(p > 0, v, x[:, REGEX_IDX:])
    fxb = jnp.concatenate([x[:, :REGEX_IDX], cell], axis=1)
    fx_ref[0] = fxb
    xl = jnp.dot(fxb, wlT_ref[0], preferred_element_type=jnp.float32) + bl_ref[0]
    xr = jnp.dot(fxb, wrT_ref[0], preferred_element_type=jnp.float32) + br_ref[0]
    one = jnp.ones((NB, 1), jnp.float32)
    z9 = jnp.zeros((NB, 9), jnp.float32)
    xl_ref[0] = jnp.concatenate([xl, one, z9], axis=1)
    xr_ref[0] = jnp.concatenate([xr, jnp.zeros((NB, 10), jnp.float32)], axis=1)
    xlL_ref[0] = xl[:, :DH]
    xlH_ref[0] = jnp.concatenate([xl[:, DH:], one, z9], axis=1)


def _compose_transform(x2, P, V, wlT2, bl2, wrT2, br2):
    p3 = P.reshape(2, N_NODES, 64)
    v3 = V.reshape(2, N_NODES, 64)
    sds = jax.ShapeDtypeStruct
    return pl.pallas_call(
        _compose_body,
        grid=(2, N_NODES // NB),
        in_specs=[
            pl.BlockSpec((1, NB, HID), lambda s, i: (s, i, 0)),
            pl.BlockSpec((1, NB, 64), lambda s, i: (s, i, 0)),
            pl.BlockSpec((1, NB, 64), lambda s, i: (s, i, 0)),
            pl.BlockSpec((1, HID, HID), lambda s, i: (s, 0, 0)),
            pl.BlockSpec((1, 1, HID), lambda s, i: (s, 0, 0)),
            pl.BlockSpec((1, HID, HID), lambda s, i: (s, 0, 0)),
            pl.BlockSpec((1, 1, HID), lambda s, i: (s, 0, 0)),
        ],
        out_specs=[
            pl.BlockSpec((1, NB, HID), lambda s, i: (s, i, 0)),
            pl.BlockSpec((1, NB, DP), lambda s, i: (s, i, 0)),
            pl.BlockSpec((1, NB, DP), lambda s, i: (s, i, 0)),
            pl.BlockSpec((1, NB, DH), lambda s, i: (s, i, 0)),
            pl.BlockSpec((1, NB, DH), lambda s, i: (s, i, 0)),
        ],
        out_shape=[
            sds((2, N_NODES, HID), jnp.float32),
            sds((2, N_NODES, DP), jnp.float32),
            sds((2, N_NODES, DP), jnp.float32),
            sds((2, N_NODES, DH), jnp.float32),
            sds((2, N_NODES, DH), jnp.float32),
        ],
    )(x2, p3, v3, wlT2, bl2, wrT2, br2)


def _k3_body(src_hbm, dst_hbm, att_hbm, xl_hbm, xr_hbm, logit_hbm, m_hbm,
             sidx0, didx0, d2idx0, zl0, zr0, sidx1, didx1, d2idx1, zl1, zr1,
             lbuf, m_local, mrg, mout, attv, sem0, sem1, mshared):
    core = lax.axis_index("c")
    sub = lax.axis_index("s")
    e0 = core * N_EDGES + sub * EPW
    pltpu.sync_copy(att_hbm.at[pl.ds(core * DP, DP)], attv)
    att_regs = [attv[pl.ds(j * 16, 16)] for j in range(DP // 16)]
    bufs = [(sidx0, didx0, d2idx0, zl0, zr0, sem0),
            (sidx1, didx1, d2idx1, zl1, zr1, sem1)]

    def initm(i, _):
        m_local[pl.ds(i * 16, 16)] = jnp.full((16,), -3e38, jnp.float32)
        return 0
    lax.fori_loop(0, N_NODES // 16, initm, 0)

    lanes = lax.iota(jnp.int32, 16)

    def start(ch, b):
        sidx, didx, d2idx, zl, zr, sem = b
        eb = e0 + ch * KCH
        pltpu.sync_copy(src_hbm.at[pl.ds(eb, KCH)], sidx)
        pltpu.sync_copy(dst_hbm.at[pl.ds(eb, KCH)], didx)
        for j in range(KCH // 16):
            sl = pl.ds(j * 16, 16)
            sidx[sl] = sidx[sl] + core * N_NODES
            d2idx[sl] = didx[sl] + core * N_NODES
        pltpu.async_copy(xl_hbm.at[sidx], zl, sem)
        pltpu.async_copy(xr_hbm.at[d2idx], zr, sem)

    def compute(ch, b):
        sidx, didx, d2idx, zl, zr, sem = b
        eb = e0 + ch * KCH
        pltpu.make_async_copy(xl_hbm.at[sidx], zl, sem).wait()
        pltpu.make_async_copy(xr_hbm.at[d2idx], zr, sem).wait()

        def edge(e, _):
            acc = jnp.zeros((16,), jnp.float32)
            for j in range(DP // 16):
                z = zl[e, pl.ds(j * 16, 16)] + zr[e, pl.ds(j * 16, 16)]
                lk = 0.6 * z + 0.4 * jnp.abs(z)
                acc = acc + lk * att_regs[j]
            lg = jnp.sum(acc)
            plsc.store_scatter(lbuf, [jnp.full((16,), 0, jnp.int32) + e],
                               jnp.zeros((16,), jnp.float32) + lg,
                               mask=lanes == 0)
            return 0
        lax.fori_loop(0, KCH, edge, 0)
        pltpu.sync_copy(lbuf, logit_hbm.at[pl.ds(eb, KCH)])

        for j in range(KCH // 16):
            sl = pl.ds(j * 16, 16)
            dv = didx[sl]
            lv = lbuf[sl]

            def mcond(st):
                return st[1] > 0

            def mbody(st):
                nv = st[0]
                cur = plsc.load_gather(m_local, [dv])
                act = nv > cur
                plsc.store_scatter(m_local, [dv], nv, mask=act)
                cur2 = plsc.load_gather(m_local, [dv])
                cnt = jnp.sum((nv > cur2).astype(jnp.int32))
                return (nv, cnt)
            lax.while_loop(mcond, mbody, (lv, jnp.int32(1)))

    start(0, bufs[0])

    def pair(p, _):
        ch = 2 * p
        start(ch + 1, bufs[1])
        compute(ch, bufs[0])
        start(ch + 2, bufs[0])
        compute(ch + 1, bufs[1])
        return 0
    lax.fori_loop(0, (EPW // KCH) // 2, pair, 0)
    compute(EPW // KCH - 1, bufs[0])

    pltpu.sync_copy(m_local, mshared.at[pl.ds(sub * NPAD, N_NODES)])
    plsc.subcore_barrier()
    for r in range(16):
        pltpu.sync_copy(mshared.at[pl.ds(r * NPAD + sub * 640, 640)],
                        mrg.at[pl.ds(r * 640, 640)])
    for j in range(40):
        a = mrg[pl.ds(j * 16, 16)]
        for rr in range(1, 16):
            a = jnp.maximum(a, mrg[pl.ds(rr * 640 + j * 16, 16)])
        a = jnp.where(a < -1e38, 0.0, a)
        mout[pl.ds(j * 16, 16)] = a
    pltpu.sync_copy(mout, m_hbm.at[pl.ds(core * NPAD + sub * 640, 640)])


def _k3_logits(src2, dst2, att2, xl4, xr4):
    mesh = plsc.VectorSubcoreMesh(core_axis_name="c", subcore_axis_name="s")
    f = pl.kernel(
        _k3_body,
        mesh=mesh,
        compiler_params=_SC_PARAMS,
        out_type=[
            jax.ShapeDtypeStruct((E2,), jnp.float32),
            jax.ShapeDtypeStruct((2 * NPAD,), jnp.float32),
        ],
        scratch_types=[
            pltpu.VMEM((KCH,), jnp.int32),
            pltpu.VMEM((KCH,), jnp.int32),
            pltpu.VMEM((KCH,), jnp.int32),
            pltpu.VMEM((KCH, DP), jnp.float32),
            pltpu.VMEM((KCH, DP), jnp.float32),
            pltpu.VMEM((KCH,), jnp.int32),
            pltpu.VMEM((KCH,), jnp.int32),
            pltpu.VMEM((KCH,), jnp.int32),
            pltpu.VMEM((KCH, DP), jnp.float32),
            pltpu.VMEM((KCH, DP), jnp.float32),
            pltpu.VMEM((KCH,), jnp.float32),
            pltpu.VMEM((N_NODES,), jnp.float32),
            pltpu.VMEM((16 * 640,), jnp.float32),
            pltpu.VMEM((640,), jnp.float32),
            pltpu.VMEM((DP,), jnp.float32),
            pltpu.SemaphoreType.DMA,
            pltpu.SemaphoreType.DMA,
            pltpu.VMEM_SHARED((16 * NPAD,), jnp.float32),
        ],
    )
    return f(src2, dst2, att2, xl4, xr4)


def _k5_body(src_hbm, dst_hbm, logit_hbm, xlL_hbm, xlH_hbm, m_hbm, out_hbm,
             sidx0, didx0, lbuf0, exbuf0, rows0, sidx1, didx1, lbuf1, exbuf1,
             rows1, m_local, zbuf, sem0, sem1, acc):
    core = lax.axis_index("c")
    sub = lax.axis_index("s")
    e0 = core * N_EDGES + sub * EPW
    pltpu.sync_copy(m_hbm.at[pl.ds(core * NPAD, N_NODES)], m_local)
    bufs = [(sidx0, didx0, lbuf0, exbuf0, rows0, sem0),
            (sidx1, didx1, lbuf1, exbuf1, rows1, sem1)]

    def zinit(i, _):
        j = i % 7
        r = i // 7
        zbuf[r, pl.ds(j * 16, 16)] = jnp.zeros((16,), jnp.float32)
        return 0
    lax.fori_loop(0, 25 * 7, zinit, 0)

    for half in range(2):
        tab = xlL_hbm if half == 0 else xlH_hbm

        def zacc(i, _):
            pltpu.sync_copy(zbuf, acc.at[pl.ds(sub * NPS + i * 25, 25)])
            return 0
        lax.fori_loop(0, NPS // 25, zacc, 0)
        plsc.subcore_barrier()

        def start(ch, b):
            sidx, didx, lbuf, exbuf, rows, sem = b
            eb = e0 + ch * KCH
            pltpu.sync_copy(src_hbm.at[pl.ds(eb, KCH)], sidx)
            pltpu.sync_copy(dst_hbm.at[pl.ds(eb, KCH)], didx)
            pltpu.sync_copy(logit_hbm.at[pl.ds(eb, KCH)], lbuf)
            for j in range(KCH // 16):
                sl = pl.ds(j * 16, 16)
                dv = didx[sl]
                mg = plsc.load_gather(m_local, [dv])
                exbuf[sl] = jnp.exp(lbuf[sl] - mg)
                sidx[sl] = sidx[sl] + core * N_NODES
            pltpu.async_copy(tab.at[sidx], rows, sem)

        def compute(b):
            sidx, didx, lbuf, exbuf, rows, sem = b
            pltpu.make_async_copy(tab.at[sidx], rows, sem).wait()

            def edge(e, _):
                w = exbuf[pl.ds(e, 16)][0]
                for j in range(DH // 16):
                    sl = pl.ds(j * 16, 16)
                    rows[e, sl] = rows[e, sl] * w
                return 0
            lax.fori_loop(0, KCH, edge, 0)
            pltpu.sync_copy(rows, acc.at[didx], add=True)

        start(0, bufs[0])

        def pair(p, _):
            start(2 * p + 1, bufs[1])
            compute(bufs[0])
            start(2 * p + 2, bufs[0])
            compute(bufs[1])
            return 0
        lax.fori_loop(0, (EPW // KCH) // 2, pair, 0)
        compute(bufs[0])
        plsc.subcore_barrier()
        pltpu.sync_copy(
            acc.at[pl.ds(sub * NPS, NPS)],
            out_hbm.at[pl.ds(half * 2 * N_NODES + core * N_NODES
                             + sub * NPS, NPS)])


def _k5_accumulate(src2, dst2, logits, xlL4, xlH4, m2):
    mesh = plsc.VectorSubcoreMesh(core_axis_name="c", subcore_axis_name="s")
    f = pl.kernel(
        _k5_body,
        mesh=mesh,
        compiler_params=_SC_PARAMS,
        out_type=jax.ShapeDtypeStruct((4 * N_NODES, DH), jnp.float32),
        scratch_types=[
            pltpu.VMEM((KCH,), jnp.int32),
            pltpu.VMEM((KCH,), jnp.int32),
            pltpu.VMEM((KCH,), jnp.float32),
            pltpu.VMEM((KCH + 16,), jnp.float32),
            pltpu.VMEM((KCH, DH), jnp.float32),
            pltpu.VMEM((KCH,), jnp.int32),
            pltpu.VMEM((KCH,), jnp.int32),
            pltpu.VMEM((KCH,), jnp.float32),
            pltpu.VMEM((KCH + 16,), jnp.float32),
            pltpu.VMEM((KCH, DH), jnp.float32),
            pltpu.VMEM((N_NODES,), jnp.float32),
            pltpu.VMEM((25, DH), jnp.float32),
            pltpu.SemaphoreType.DMA,
            pltpu.SemaphoreType.DMA,
            pltpu.VMEM_SHARED((N_NODES, DH), jnp.float32),
        ],
    )
    return f(src2, dst2, logits, xlL4, xlH4, m2)


def _final_body(accL_ref, accH_ref, fx_ref, bias_ref, out_ref):
    accL = accL_ref[0]
    accH = accH_ref[0]
    den = accH[:, 102:103] + 1e-16
    num = jnp.concatenate([accL, accH[:, :102]], axis=1)
    out_ref[0] = jax.nn.relu(num / den + bias_ref[0] + fx_ref[0])


def _final(accL, accH, fx, bias2):
    out = pl.pallas_call(
        _final_body,
        grid=(2, N_NODES // NB),
        in_specs=[
            pl.BlockSpec((1, NB, DH), lambda s, i: (s, i, 0)),
            pl.BlockSpec((1, NB, DH), lambda s, i: (s, i, 0)),
            pl.BlockSpec((1, NB, HID), lambda s, i: (s, i, 0)),
            pl.BlockSpec((1, 1, HID), lambda s, i: (s, 0, 0)),
        ],
        out_specs=pl.BlockSpec((1, NB, HID), lambda s, i: (s, i, 0)),
        out_shape=jax.ShapeDtypeStruct((2, N_NODES, HID), jnp.float32),
    )(accL, accH, fx, bias2)
    return out


def kernel(fwd_x, fwd_edge_index, fwd_edge_attr, bwd_x, bwd_edge_index,
           bwd_edge_attr, embed, Wih_f, Whh_f, bih_f, bhh_f, Wih_r, Whh_r,
           bih_r, bhh_r, lin1_W, lin1_b, lin2_W, lin2_b, fgat_Wl, fgat_bl,
           fgat_Wr, fgat_br, fgat_att, fgat_bias, bgat_Wl, bgat_bl, bgat_Wr,
           bgat_br, bgat_att, bgat_bias):
    tokens2 = jnp.concatenate([fwd_edge_attr, bwd_edge_attr], axis=0)
    vals2 = _edge_scores(tokens2, embed, Wih_f, Whh_f, bih_f, bhh_f,
                         Wih_r, bih_r, bhh_r, lin1_W, lin1_b, lin2_W, lin2_b)

    f_src, f_dst = fwd_edge_index[0], fwd_edge_index[1]
    b_src, b_dst = bwd_edge_index[0], bwd_edge_index[1]
    x2 = jnp.stack([fwd_x, bwd_x])
    tid2 = _tid_argmax(x2)

    rows2 = jnp.concatenate([f_src, b_dst]).astype(jnp.int32)
    src2 = jnp.concatenate([f_src, b_src]).astype(jnp.int32)
    dst2 = jnp.concatenate([f_dst, b_dst]).astype(jnp.int32)
    P, V = _k1_overwrite_winner(rows2, dst2, vals2, tid2)

    wlT2 = jnp.stack([fgat_Wl.T, bgat_Wl.T])
    wrT2 = jnp.stack([fgat_Wr.T, bgat_Wr.T])
    bl2 = jnp.stack([fgat_bl, bgat_bl])[:, None, :]
    br2 = jnp.stack([fgat_br, bgat_br])[:, None, :]
    fx2, xl2, xr2, xlL2, xlH2 = _compose_transform(x2, P, V, wlT2, bl2,
                                                   wrT2, br2)

    att2 = jnp.concatenate([
        jnp.pad(fgat_att, (0, DP - HID)), jnp.pad(bgat_att, (0, DP - HID))])
    xl4 = xl2.reshape(2 * N_NODES, DP)
    xr4 = xr2.reshape(2 * N_NODES, DP)
    logits, m2 = _k3_logits(src2, dst2, att2, xl4, xr4)

    xlL4 = xlL2.reshape(2 * N_NODES, DH)
    xlH4 = xlH2.reshape(2 * N_NODES, DH)
    accs = _k5_accumulate(src2, dst2, logits, xlL4, xlH4, m2)
    accL = accs[:2 * N_NODES].reshape(2, N_NODES, DH)
    accH = accs[2 * N_NODES:].reshape(2, N_NODES, DH)

    bias2 = jnp.stack([fgat_bias, bgat_bias])[:, None, :]
    out2 = _final(accL, accH, fx2, bias2)
    return jnp.concatenate([out2[0], out2[1]], axis=1)


# speedup vs baseline: 54.4490x; 1.0565x over previous
import functools

import jax
import jax.numpy as jnp
from jax import lax
from jax.experimental import pallas as pl
from jax.experimental.pallas import tpu as pltpu
from jax.experimental.pallas import tpu_sc as plsc

MAX_STATES = 50
TID = MAX_STATES + 3
REGEX_IDX = TID + 2 + TID + TID
HID = REGEX_IDX + TID
N_NODES = 10000
N_EDGES = 160000
SEQ_LEN = 8
VOCAB = 100
EMB = 32
LSTM = 64
H4 = 4 * LSTM

EB = 2560
E2 = 2 * N_EDGES
NBLK = E2 // EB

NKEY = N_NODES * 64
KW = NKEY // 16
ECH = 4000

NB = 1000
DP = 224
DH = 112
EPW = N_EDGES // 16
KCH = 80
NPS = N_NODES // 16
NPAD = 10240

_SC_PARAMS = pltpu.CompilerParams(needs_layout_passes=False,
                                  use_tc_tiling_on_sc=False)


def _edge_score_body(tok_ref, embT_ref, wihf_ref, whhf_ref, bf_ref,
                     wihr_ref, br_ref, l1w_ref, l1b_ref, l2w_ref, l2b_ref,
                     out_ref):
    tok = tok_ref[...]
    embT = embT_ref[...]
    Af = jnp.dot(wihf_ref[...], embT, preferred_element_type=jnp.float32)
    Whh = whhf_ref[...]
    bf = bf_ref[...]

    def onehot(row):
        i = jax.lax.broadcasted_iota(jnp.int32, (VOCAB, EB), 0)
        return (i == row[None, :]).astype(jnp.float32)

    sig = jax.nn.sigmoid
    tnh = jnp.tanh

    h = jnp.zeros((LSTM, EB), jnp.float32)
    c = jnp.zeros((LSTM, EB), jnp.float32)
    for t in range(SEQ_LEN):
        oh = onehot(tok[t])
        g = (jnp.dot(Af, oh, preferred_element_type=jnp.float32)
             + jnp.dot(Whh, h, preferred_element_type=jnp.float32) + bf)
        i_g = sig(g[0:LSTM])
        f_g = sig(g[LSTM:2 * LSTM])
        gg = tnh(g[2 * LSTM:3 * LSTM])
        o_g = sig(g[3 * LSTM:4 * LSTM])
        c = f_g * c + i_g * gg
        h = o_g * tnh(c)

    Ar = jnp.dot(wihr_ref[...], embT, preferred_element_type=jnp.float32)
    gr = jnp.dot(Ar, onehot(tok[SEQ_LEN - 1]),
                 preferred_element_type=jnp.float32) + br_ref[...]
    c_r = sig(gr[0:LSTM]) * tnh(gr[2 * LSTM:3 * LSTM])
    h_r = sig(gr[3 * LSTM:4 * LSTM]) * tnh(c_r)

    hcat = jnp.concatenate([h, h_r], axis=0)
    v = jax.nn.relu(jnp.dot(l1w_ref[...], hcat,
                            preferred_element_type=jnp.float32) + l1b_ref[...])
    s = jax.nn.relu(jnp.dot(l2w_ref[...], v,
                            preferred_element_type=jnp.float32) + l2b_ref[...])
    out_ref[0, 0, :] = s[0]


def _edge_scores(tokens2, embed, Wih_f, Whh_f, bih_f, bhh_f,
                 Wih_r, bih_r, bhh_r, lin1_W, lin1_b, lin2_W, lin2_b):
    tokT = tokens2.T.astype(jnp.int32)
    embT = embed.at[0].set(0.0).T
    bf = (bih_f + bhh_f)[:, None]
    br = (bih_r + bhh_r)[:, None]
    full = lambda shape: pl.BlockSpec(shape, lambda i: (0,) * len(shape))
    out = pl.pallas_call(
        _edge_score_body,
        grid=(NBLK,),
        in_specs=[
            pl.BlockSpec((SEQ_LEN, EB), lambda i: (0, i)),
            full((EMB, VOCAB)),
            full((H4, EMB)),
            full((H4, LSTM)),
            full((H4, 1)),
            full((H4, EMB)),
            full((H4, 1)),
            full((32, 2 * LSTM)),
            full((32, 1)),
            full((1, 32)),
            full((1, 1)),
        ],
        out_specs=pl.BlockSpec((1, 1, EB), lambda i: (i, 0, 0)),
        out_shape=jax.ShapeDtypeStruct((NBLK, 1, EB), jnp.float32),
    )(tokT, embT, Wih_f, Whh_f, bf, Wih_r, br,
      lin1_W, lin1_b[:, None], lin2_W, lin2_b[:, None])
    return out.reshape(E2)


def _tid_body(x_ref, out_ref):
    t = x_ref[0][:, :TID]
    m = jnp.max(t, axis=1, keepdims=True)
    iota = jax.lax.broadcasted_iota(jnp.int32, (NB, TID), 1)
    idx = jnp.min(jnp.where(t == m, iota, TID), axis=1, keepdims=True)
    out_ref[0] = jnp.broadcast_to(idx, (NB, 8))


def _tid_argmax(x2):
    out = pl.pallas_call(
        _tid_body,
        grid=(2, N_NODES // NB),
        in_specs=[pl.BlockSpec((1, NB, HID), lambda s, i: (s, i, 0))],
        out_specs=pl.BlockSpec((1, NB, 8), lambda s, i: (s, i, 0)),
        out_shape=jax.ShapeDtypeStruct((2, N_NODES, 8), jnp.int32),
    )(x2)
    return out[:, :, 0].reshape(2 * N_NODES)


def _k1_body(rows_hbm, dst_hbm, vals_hbm, tid_hbm, p_hbm, v_hbm,
             tidv, rbuf0, dbuf0, vbuf0, rbuf1, dbuf1, vbuf1, pv, vv, tmp,
             sem0, sem1):
    core = lax.axis_index("c")
    sub = lax.axis_index("s")
    base = sub * KW
    e0 = core * N_EDGES
    pltpu.sync_copy(tid_hbm.at[pl.ds(core * N_NODES, N_NODES)], tidv)
    bufs = [(rbuf0, dbuf0, vbuf0, sem0), (rbuf1, dbuf1, vbuf1, sem1)]
    nch = N_EDGES // ECH

    def zero(i, _):
        pv[pl.ds(i * 16, 16)] = jnp.zeros((16,), jnp.int32)
        return 0
    lax.fori_loop(0, KW // 16, zero, 0)

    lanes = lax.iota(jnp.int32, 16)
    tmp[pl.ds(16, 16)] = jnp.full((16,), -1, jnp.int32)
    ones = jnp.ones((16,), jnp.int32)

    def startc(i, b):
        rb, db, vb, sem = b
        pltpu.async_copy(rows_hbm.at[pl.ds(e0 + i * ECH, ECH)], rb, sem)
        pltpu.async_copy(dst_hbm.at[pl.ds(e0 + i * ECH, ECH)], db, sem)
        pltpu.async_copy(vals_hbm.at[pl.ds(e0 + i * ECH, ECH)], vb, sem)

    def computec(b):
        rb, db, vb, sem = b
        pltpu.make_async_copy(rows_hbm.at[pl.ds(e0, ECH)], rb, sem).wait()
        pltpu.make_async_copy(dst_hbm.at[pl.ds(e0, ECH)], db, sem).wait()
        pltpu.make_async_copy(vals_hbm.at[pl.ds(e0, ECH)], vb, sem).wait()

        def step(j, _):
            r = rb[pl.ds(j * 16, 16)]
            d = db[pl.ds(j * 16, 16)]
            v = vb[pl.ds(j * 16, 16)]
            t = plsc.load_gather(tidv, [d])
            key = r * 64 + t
            skey = key * 16 + lanes
            ks, vs = plsc.sort_key_val(skey, v)
            tmp[pl.ds(0, 16)] = ks
            nx = plsc.load_gather(tmp, [lanes + 1])
            kq = lax.shift_right_logical(ks, 4)
            nq = lax.shift_right_logical(nx, 4)
            msk = (kq != nq) & (kq >= base) & (kq < base + KW)
            loc = jnp.clip(kq - base, 0, KW - 1)
            plsc.store_scatter(vv, [loc], vs, mask=msk)
            plsc.store_scatter(pv, [loc], ones, mask=msk)
            return 0
        lax.fori_loop(0, ECH // 16, step, 0)

    startc(0, bufs[0])

    def pair(p, _):
        startc(2 * p + 1, bufs[1])
        computec(bufs[0])

        @pl.when(2 * p + 2 < nch)
        def _():
            startc(2 * p + 2, bufs[0])
        computec(bufs[1])
        return 0
    lax.fori_loop(0, nch // 2, pair, 0)

    pltpu.sync_copy(pv, p_hbm.at[pl.ds(core * NKEY + base, KW)])
    pltpu.sync_copy(vv, v_hbm.at[pl.ds(core * NKEY + base, KW)])


def _k1_overwrite_winner(rows2, dst2, vals2, tid2):
    mesh = plsc.VectorSubcoreMesh(core_axis_name="c", subcore_axis_name="s")
    f = pl.kernel(
        _k1_body,
        mesh=mesh,
        compiler_params=_SC_PARAMS,
        out_type=[
            jax.ShapeDtypeStruct((2 * NKEY,), jnp.int32),
            jax.ShapeDtypeStruct((2 * NKEY,), jnp.float32),
        ],
        scratch_types=[
            pltpu.VMEM((N_NODES,), jnp.int32),
            pltpu.VMEM((ECH,), jnp.int32),
            pltpu.VMEM((ECH,), jnp.int32),
            pltpu.VMEM((ECH,), jnp.float32),
            pltpu.VMEM((ECH,), jnp.int32),
            pltpu.VMEM((ECH,), jnp.int32),
            pltpu.VMEM((ECH,), jnp.float32),
            pltpu.VMEM((KW,), jnp.int32),
            pltpu.VMEM((KW,), jnp.float32),
            pltpu.VMEM((32,), jnp.int32),
            pltpu.SemaphoreType.DMA,
            pltpu.SemaphoreType.DMA,
        ],
    )
    return f(rows2, dst2, vals2, tid2)


def _compose_body(x_ref, p_ref, v_ref, wlT_ref, bl_ref, wrT_ref, br_ref,
                  fx_ref, xl_ref, xr_ref, xlL_ref, xlH_ref):
    x = x_ref[0]
    p = p_ref[0][:, :TID]
    v = v_ref[0][:, :TID]
    cell = jnp.where(p > 0, v, x[:, REGEX_IDX:])
    fxb = jnp.concatenate([x[:, :REGEX_IDX], cell], axis=1)
    fx_ref[...] = fxb
    xl = jnp.dot(fxb, wlT_ref[0], preferred_element_type=jnp.float32) + bl_ref[0]
    xr = jnp.dot(fxb, wrT_ref[0], preferred_element_type=jnp.float32) + br_ref[0]
    one = jnp.ones((NB, 1), jnp.float32)
    z9 = jnp.zeros((NB, 9), jnp.float32)
    xl_ref[...] = jnp.concatenate([xl, one, z9], axis=1)
    xr_ref[...] = jnp.concatenate([xr, jnp.zeros((NB, 10), jnp.float32)],
                                  axis=1)
    xlL_ref[...] = xl[:, :DH]
    xlH_ref[...] = jnp.concatenate([xl[:, DH:], one, z9], axis=1)


def _compose_transform(x2, P, V, wlT2, bl2, wrT2, br2):
    p3 = P.reshape(2, N_NODES, 64)
    v3 = V.reshape(2, N_NODES, 64)
    sds = jax.ShapeDtypeStruct
    row = lambda s, i: (s * (N_NODES // NB) + i, 0)
    return pl.pallas_call(
        _compose_body,
        grid=(2, N_NODES // NB),
        in_specs=[
            pl.BlockSpec((1, NB, HID), lambda s, i: (s, i, 0)),
            pl.BlockSpec((1, NB, 64), lambda s, i: (s, i, 0)),
            pl.BlockSpec((1, NB, 64), lambda s, i: (s, i, 0)),
            pl.BlockSpec((1, HID, HID), lambda s, i: (s, 0, 0)),
            pl.BlockSpec((1, 1, HID), lambda s, i: (s, 0, 0)),
            pl.BlockSpec((1, HID, HID), lambda s, i: (s, 0, 0)),
            pl.BlockSpec((1, 1, HID), lambda s, i: (s, 0, 0)),
        ],
        out_specs=[
            pl.BlockSpec((NB, HID), row),
            pl.BlockSpec((NB, DP), row),
            pl.BlockSpec((NB, DP), row),
            pl.BlockSpec((NB, DH), row),
            pl.BlockSpec((NB, DH), row),
        ],
        out_shape=[
            sds((2 * N_NODES, HID), jnp.float32),
            sds((2 * N_NODES, DP), jnp.float32),
            sds((2 * N_NODES, DP), jnp.float32),
            sds((2 * N_NODES, DH), jnp.float32),
            sds((2 * N_NODES, DH), jnp.float32),
        ],
    )(x2, p3, v3, wlT2, bl2, wrT2, br2)


def _k3_body(src_hbm, dst_hbm, att_hbm, xl_hbm, xr_hbm, logit_hbm, m_hbm,
             sidx0, didx0, d2idx0, zl0, zr0, sidx1, didx1, d2idx1, zl1, zr1,
             lbuf, m_local, mrg, mout, attv, sem0, sem1, mshared):
    core = lax.axis_index("c")
    sub = lax.axis_index("s")
    e0 = core * N_EDGES + sub * EPW
    pltpu.sync_copy(att_hbm.at[pl.ds(core * DP, DP)], attv)
    att_regs = [attv[pl.ds(j * 16, 16)] for j in range(DP // 16)]
    bufs = [(sidx0, didx0, d2idx0, zl0, zr0, sem0),
            (sidx1, didx1, d2idx1, zl1, zr1, sem1)]

    def initm(i, _):
        m_local[pl.ds(i * 16, 16)] = jnp.full((16,), -3e38, jnp.float32)
        return 0
    lax.fori_loop(0, N_NODES // 16, initm, 0)

    lanes = lax.iota(jnp.int32, 16)

    def start(ch, b):
        sidx, didx, d2idx, zl, zr, sem = b
        eb = e0 + ch * KCH
        pltpu.sync_copy(src_hbm.at[pl.ds(eb, KCH)], sidx)
        pltpu.sync_copy(dst_hbm.at[pl.ds(eb, KCH)], didx)
        for j in range(KCH // 16):
            sl = pl.ds(j * 16, 16)
            sidx[sl] = sidx[sl] + core * N_NODES
            d2idx[sl] = didx[sl] + core * N_NODES
        pltpu.async_copy(xl_hbm.at[sidx], zl, sem)
        pltpu.async_copy(xr_hbm.at[d2idx], zr, sem)

    def compute(ch, b):
        sidx, didx, d2idx, zl, zr, sem = b
        eb = e0 + ch * KCH
        pltpu.make_async_copy(xl_hbm.at[sidx], zl, sem).wait()
        pltpu.make_async_copy(xr_hbm.at[d2idx], zr, sem).wait()

        def edge(e, _):
            acc = jnp.zeros((16,), jnp.float32)
            for j in range(DP // 16):
                z = zl[e, pl.ds(j * 16, 16)] + zr[e, pl.ds(j * 16, 16)]
                lk = 0.6 * z + 0.4 * jnp.abs(z)
                acc = acc + lk * att_regs[j]
            lg = jnp.sum(acc)
            plsc.store_scatter(lbuf, [jnp.full((16,), 0, jnp.int32) + e],
                               jnp.zeros((16,), jnp.float32) + lg,
                               mask=lanes == 0)
            return 0
        lax.fori_loop(0, KCH, edge, 0)
        pltpu.sync_copy(lbuf, logit_hbm.at[pl.ds(eb, KCH)])

        for j in range(KCH // 16):
            sl = pl.ds(j * 16, 16)
            dv = didx[sl]
            lv = lbuf[sl]

            def mcond(st):
                return st[1] > 0

            def mbody(st):
                nv = st[0]
                cur = plsc.load_gather(m_local, [dv])
                act = nv > cur
                plsc.store_scatter(m_local, [dv], nv, mask=act)
                cur2 = plsc.load_gather(m_local, [dv])
                cnt = jnp.sum((nv > cur2).astype(jnp.int32))
                return (nv, cnt)
            lax.while_loop(mcond, mbody, (lv, jnp.int32(1)))

    start(0, bufs[0])

    def pair(p, _):
        ch = 2 * p
        start(ch + 1, bufs[1])
        compute(ch, bufs[0])
        start(ch + 2, bufs[0])
        compute(ch + 1, bufs[1])
        return 0
    lax.fori_loop(0, (EPW // KCH) // 2, pair, 0)
    compute(EPW // KCH - 1, bufs[0])

    pltpu.sync_copy(m_local, mshared.at[pl.ds(sub * NPAD, N_NODES)])
    plsc.subcore_barrier()
    for r in range(16):
        pltpu.sync_copy(mshared.at[pl.ds(r * NPAD + sub * 640, 640)],
                        mrg.at[pl.ds(r * 640, 640)])
    for j in range(40):
        a = mrg[pl.ds(j * 16, 16)]
        for rr in range(1, 16):
            a = jnp.maximum(a, mrg[pl.ds(rr * 640 + j * 16, 16)])
        a = jnp.where(a < -1e38, 0.0, a)
        mout[pl.ds(j * 16, 16)] = a
    pltpu.sync_copy(mout, m_hbm.at[pl.ds(core * NPAD + sub * 640, 640)])


def _k3_logits(src2, dst2, att2, xl4, xr4):
    mesh = plsc.VectorSubcoreMesh(core_axis_name="c", subcore_axis_name="s")
    f = pl.kernel(
        _k3_body,
        mesh=mesh,
        compiler_params=_SC_PARAMS,
        out_type=[
            jax.ShapeDtypeStruct((E2,), jnp.float32),
            jax.ShapeDtypeStruct((2 * NPAD,), jnp.float32),
        ],
        scratch_types=[
            pltpu.VMEM((KCH,), jnp.int32),
            pltpu.VMEM((KCH,), jnp.int32),
            pltpu.VMEM((KCH,), jnp.int32),
            pltpu.VMEM((KCH, DP), jnp.float32),
            pltpu.VMEM((KCH, DP), jnp.float32),
            pltpu.VMEM((KCH,), jnp.int32),
            pltpu.VMEM((KCH,), jnp.int32),
            pltpu.VMEM((KCH,), jnp.int32),
            pltpu.VMEM((KCH, DP), jnp.float32),
            pltpu.VMEM((KCH, DP), jnp.float32),
            pltpu.VMEM((KCH,), jnp.float32),
            pltpu.VMEM((N_NODES,), jnp.float32),
            pltpu.VMEM((16 * 640,), jnp.float32),
            pltpu.VMEM((640,), jnp.float32),
            pltpu.VMEM((DP,), jnp.float32),
            pltpu.SemaphoreType.DMA,
            pltpu.SemaphoreType.DMA,
            pltpu.VMEM_SHARED((16 * NPAD,), jnp.float32),
        ],
    )
    return f(src2, dst2, att2, xl4, xr4)


def _k5_body(src_hbm, dst_hbm, logit_hbm, xlL_hbm, xlH_hbm, m_hbm, out_hbm,
             sidx0, didx0, lbuf0, exbuf0, rows0, sidx1, didx1, lbuf1, exbuf1,
             rows1, m_local, zbuf, sem0, sem1, acc):
    core = lax.axis_index("c")
    sub = lax.axis_index("s")
    e0 = core * N_EDGES + sub * EPW
    pltpu.sync_copy(m_hbm.at[pl.ds(core * NPAD, N_NODES)], m_local)
    bufs = [(sidx0, didx0, lbuf0, exbuf0, rows0, sem0),
            (sidx1, didx1, lbuf1, exbuf1, rows1, sem1)]

    def zinit(i, _):
        j = i % 7
        r = i // 7
        zbuf[r, pl.ds(j * 16, 16)] = jnp.zeros((16,), jnp.float32)
        return 0
    lax.fori_loop(0, 25 * 7, zinit, 0)

    for half in range(2):
        tab = xlL_hbm if half == 0 else xlH_hbm

        def zacc(i, _):
            pltpu.sync_copy(zbuf, acc.at[pl.ds(sub * NPS + i * 25, 25)])
            return 0
        lax.fori_loop(0, NPS // 25, zacc, 0)
        plsc.subcore_barrier()

        def start(ch, b):
            sidx, didx, lbuf, exbuf, rows, sem = b
            eb = e0 + ch * KCH
            pltpu.sync_copy(src_hbm.at[pl.ds(eb, KCH)], sidx)
            pltpu.sync_copy(dst_hbm.at[pl.ds(eb, KCH)], didx)
            pltpu.sync_copy(logit_hbm.at[pl.ds(eb, KCH)], lbuf)
            for j in range(KCH // 16):
                sl = pl.ds(j * 16, 16)
                dv = didx[sl]
                mg = plsc.load_gather(m_local, [dv])
                exbuf[sl] = jnp.exp(lbuf[sl] - mg)
                sidx[sl] = sidx[sl] + core * N_NODES
            pltpu.async_copy(tab.at[sidx], rows, sem)

        def compute(b):
            sidx, didx, lbuf, exbuf, rows, sem = b
            pltpu.make_async_copy(tab.at[sidx], rows, sem).wait()

            def edge(e, _):
                w = exbuf[pl.ds(e, 16)][0]
                for j in range(DH // 16):
                    sl = pl.ds(j * 16, 16)
                    rows[e, sl] = rows[e, sl] * w
                return 0
            lax.fori_loop(0, KCH, edge, 0)
            pltpu.sync_copy(rows, acc.at[didx], add=True)

        start(0, bufs[0])

        def pair(p, _):
            start(2 * p + 1, bufs[1])
            compute(bufs[0])
            start(2 * p + 2, bufs[0])
            compute(bufs[1])
            return 0
        lax.fori_loop(0, (EPW // KCH) // 2, pair, 0)
        compute(bufs[0])
        plsc.subcore_barrier()
        pltpu.sync_copy(
            acc.at[pl.ds(sub * NPS, NPS)],
            out_hbm.at[pl.ds(half * 2 * N_NODES + core * N_NODES
                             + sub * NPS, NPS)])


def _k5_accumulate(src2, dst2, logits, xlL4, xlH4, m2):
    mesh = plsc.VectorSubcoreMesh(core_axis_name="c", subcore_axis_name="s")
    f = pl.kernel(
        _k5_body,
        mesh=mesh,
        compiler_params=_SC_PARAMS,
        out_type=jax.ShapeDtypeStruct((4 * N_NODES, DH), jnp.float32),
        scratch_types=[
            pltpu.VMEM((KCH,), jnp.int32),
            pltpu.VMEM((KCH,), jnp.int32),
            pltpu.VMEM((KCH,), jnp.float32),
            pltpu.VMEM((KCH + 16,), jnp.float32),
            pltpu.VMEM((KCH, DH), jnp.float32),
            pltpu.VMEM((KCH,), jnp.int32),
            pltpu.VMEM((KCH,), jnp.int32),
            pltpu.VMEM((KCH,), jnp.float32),
            pltpu.VMEM((KCH + 16,), jnp.float32),
            pltpu.VMEM((KCH, DH), jnp.float32),
            pltpu.VMEM((N_NODES,), jnp.float32),
            pltpu.VMEM((25, DH), jnp.float32),
            pltpu.SemaphoreType.DMA,
            pltpu.SemaphoreType.DMA,
            pltpu.VMEM_SHARED((N_NODES, DH), jnp.float32),
        ],
    )
    return f(src2, dst2, logits, xlL4, xlH4, m2)


def _final_body(accL_ref, accH_ref, fx_ref, bias_ref, out_ref):
    accL = accL_ref[...]
    accH = accH_ref[...]
    den = accH[:, 102:103] + 1e-16
    num = jnp.concatenate([accL, accH[:, :102]], axis=1)
    out_ref[0] = jax.nn.relu(num / den + bias_ref[0] + fx_ref[...])


def _final(accs, fx, bias2):
    nb = N_NODES // NB
    row = lambda s, i: (s * nb + i, 0)
    out = pl.pallas_call(
        _final_body,
        grid=(2, nb),
        in_specs=[
            pl.BlockSpec((NB, DH), row),
            pl.BlockSpec((NB, DH), lambda s, i: (2 * nb + s * nb + i, 0)),
            pl.BlockSpec((NB, HID), row),
            pl.BlockSpec((1, 1, HID), lambda s, i: (s, 0, 0)),
        ],
        out_specs=pl.BlockSpec((1, NB, HID), lambda s, i: (s, i, 0)),
        out_shape=jax.ShapeDtypeStruct((2, N_NODES, HID), jnp.float32),
    )(accs, accs, fx, bias2)
    return out


def kernel(fwd_x, fwd_edge_index, fwd_edge_attr, bwd_x, bwd_edge_index,
           bwd_edge_attr, embed, Wih_f, Whh_f, bih_f, bhh_f, Wih_r, Whh_r,
           bih_r, bhh_r, lin1_W, lin1_b, lin2_W, lin2_b, fgat_Wl, fgat_bl,
           fgat_Wr, fgat_br, fgat_att, fgat_bias, bgat_Wl, bgat_bl, bgat_Wr,
           bgat_br, bgat_att, bgat_bias):
    tokens2 = jnp.concatenate([fwd_edge_attr, bwd_edge_attr], axis=0)
    vals2 = _edge_scores(tokens2, embed, Wih_f, Whh_f, bih_f, bhh_f,
                         Wih_r, bih_r, bhh_r, lin1_W, lin1_b, lin2_W, lin2_b)

    f_src, f_dst = fwd_edge_index[0], fwd_edge_index[1]
    b_src, b_dst = bwd_edge_index[0], bwd_edge_index[1]
    x2 = jnp.stack([fwd_x, bwd_x])
    tid2 = _tid_argmax(x2)

    rows2 = jnp.concatenate([f_src, b_dst]).astype(jnp.int32)
    src2 = jnp.concatenate([f_src, b_src]).astype(jnp.int32)
    dst2 = jnp.concatenate([f_dst, b_dst]).astype(jnp.int32)
    P, V = _k1_overwrite_winner(rows2, dst2, vals2, tid2)

    wlT2 = jnp.stack([fgat_Wl.T, bgat_Wl.T])
    wrT2 = jnp.stack([fgat_Wr.T, bgat_Wr.T])
    bl2 = jnp.stack([fgat_bl, bgat_bl])[:, None, :]
    br2 = jnp.stack([fgat_br, bgat_br])[:, None, :]
    fx2, xl2, xr2, xlL2, xlH2 = _compose_transform(x2, P, V, wlT2, bl2,
                                                   wrT2, br2)

    att2 = jnp.concatenate([
        jnp.pad(fgat_att, (0, DP - HID)), jnp.pad(bgat_att, (0, DP - HID))])
    logits, m2 = _k3_logits(src2, dst2, att2, xl2, xr2)

    accs = _k5_accumulate(src2, dst2, logits, xlL2, xlH2, m2)

    bias2 = jnp.stack([fgat_bias, bgat_bias])[:, None, :]
    out2 = _final(accs, fx2, bias2)
    return jnp.concatenate([out2[0], out2[1]], axis=1)
